# baseline (speedup 1.0000x reference)
import functools

import jax
import jax.numpy as jnp
from jax import lax
from jax.experimental import pallas as pl
from jax.experimental.pallas import tpu as pltpu
from jax.experimental.pallas import tpu_sc as plsc

NC = 2
NS = 16
NW = NC * NS
NBUF = 4


@functools.partial(jax.jit, static_argnames=("bw", "hist", "d"))
def _sc_gather(idx, weight, *, bw, hist, d):
  outer = bw // NBUF
  mesh = plsc.VectorSubcoreMesh(
      core_axis_name="c", subcore_axis_name="s",
      num_cores=NC, num_subcores=NS)

  @functools.partial(
      pl.kernel,
      out_type=jax.ShapeDtypeStruct((NW * bw, hist, d), jnp.float32),
      mesh=mesh,
      scratch_types=[
          pltpu.VMEM((bw, hist), jnp.int32),
          pltpu.VMEM((NBUF, hist, d), jnp.float32),
          pltpu.SemaphoreType.DMA((NBUF,)),
          pltpu.SemaphoreType.DMA((NBUF,)),
      ],
      compiler_params=pltpu.CompilerParams(use_tc_tiling_on_sc=False),
  )
  def body(idx_hbm, table_hbm, out_hbm, idx_v, rows_v, sem_in, sem_out):
    wid = lax.axis_index("s") * NC + lax.axis_index("c")
    pltpu.sync_copy(idx_hbm.at[pl.ds(wid * bw, bw)], idx_v)
    for b in range(NBUF):
      pltpu.async_copy(table_hbm.at[idx_v.at[b]], rows_v.at[b], sem_in.at[b])

    @pl.loop(0, outer)
    def _(g):
      for b in range(NBUF):
        r = g * NBUF + b
        pltpu.make_async_copy(
            table_hbm.at[idx_v.at[r]], rows_v.at[b], sem_in.at[b]).wait()
        pltpu.async_copy(rows_v.at[b], out_hbm.at[wid * bw + r], sem_out.at[b])

        @pl.when(g < outer - 1)
        def _():
          pltpu.make_async_copy(
              rows_v.at[b], out_hbm.at[wid * bw + r], sem_out.at[b]).wait()
          pltpu.async_copy(
              table_hbm.at[idx_v.at[r + NBUF]], rows_v.at[b], sem_in.at[b])

    for b in range(NBUF):
      r = (outer - 1) * NBUF + b
      pltpu.make_async_copy(
          rows_v.at[b], out_hbm.at[wid * bw + r], sem_out.at[b]).wait()

  return body(idx, weight)


def kernel(inputs, weight):
  batch, hist = inputs.shape
  d = weight.shape[1]
  assert batch % (NW * NBUF) == 0
  bw = batch // NW
  idx = inputs.astype(jnp.int32)
  return _sc_gather(idx, weight, bw=bw, hist=hist, d=d)


# speedup vs baseline: 1.2533x; 1.2533x over previous
import functools

import jax
import jax.numpy as jnp
from jax import lax
from jax.experimental.layout import Format, Layout
from jax.experimental import pallas as pl
from jax.experimental.pallas import tpu as pltpu
from jax.experimental.pallas import tpu_sc as plsc

NC = 2
NS = 16
NW = NC * NS
NBUF = 4


@functools.partial(jax.jit, static_argnames=("bw", "hist", "d"))
def _sc_gather(idx, weight, *, bw, hist, d):
  outer = bw // NBUF
  mesh = plsc.VectorSubcoreMesh(
      core_axis_name="c", subcore_axis_name="s",
      num_cores=NC, num_subcores=NS)

  @functools.partial(
      pl.kernel,
      out_type=jax.ShapeDtypeStruct((NW * bw, hist, d), jnp.float32),
      mesh=mesh,
      scratch_types=[
          pltpu.VMEM((bw, hist), jnp.int32),
          pltpu.VMEM((NBUF, hist, d), jnp.float32),
          pltpu.SemaphoreType.DMA((NBUF,)),
          pltpu.SemaphoreType.DMA((NBUF,)),
      ],
      compiler_params=pltpu.CompilerParams(use_tc_tiling_on_sc=False),
  )
  def body(idx_hbm, table_hbm, out_hbm, idx_v, rows_v, sem_in, sem_out):
    wid = lax.axis_index("s") * NC + lax.axis_index("c")
    pltpu.sync_copy(idx_hbm.at[pl.ds(wid * bw, bw)], idx_v)
    for b in range(NBUF):
      pltpu.async_copy(table_hbm.at[idx_v.at[b]], rows_v.at[b], sem_in.at[b])

    @pl.loop(0, outer)
    def _(g):
      for b in range(NBUF):
        r = g * NBUF + b
        pltpu.make_async_copy(
            table_hbm.at[idx_v.at[r]], rows_v.at[b], sem_in.at[b]).wait()
        pltpu.async_copy(rows_v.at[b], out_hbm.at[wid * bw + r], sem_out.at[b])

        @pl.when(g < outer - 1)
        def _():
          pltpu.make_async_copy(
              rows_v.at[b], out_hbm.at[wid * bw + r], sem_out.at[b]).wait()
          pltpu.async_copy(
              table_hbm.at[idx_v.at[r + NBUF]], rows_v.at[b], sem_in.at[b])

    for b in range(NBUF):
      r = (outer - 1) * NBUF + b
      pltpu.make_async_copy(
          rows_v.at[b], out_hbm.at[wid * bw + r], sem_out.at[b]).wait()

  return body(idx, weight)


def kernel(inputs, weight):
  batch, hist = inputs.shape
  d = weight.shape[1]
  assert batch % (NW * NBUF) == 0
  bw = batch // NW
  idx = inputs.astype(jnp.int32)
  from jax.experimental.layout import with_layout_constraint
  wt = with_layout_constraint(
      weight, Layout(major_to_minor=(0, 1), tiling=((8,),)))
  return _sc_gather(idx, wt, bw=bw, hist=hist, d=d)
